# baseline (speedup 1.0000x reference)
import functools

import jax
import jax.numpy as jnp
from jax import lax
from jax.experimental import pallas as pl
from jax.experimental.pallas import tpu as pltpu
from jax.experimental.pallas import tpu_sc as plsc

N_NODES = 10000
N_EDGES = 320000
HID = 128
NUM_GRAPHS = 64
NUM_LAYERS = 4
NUM_TIMESTEPS = 16

NP = 10240
BLK = 1024
EPS = 1e-16

NW = 32
EP = 327680
TILE_E = EP // NW
EC = 512
ROWS_T = NP // 16


def _leaky(v):
    return jnp.where(v > 0, v, 0.01 * v)


_iota16 = lambda: lax.broadcasted_iota(jnp.int32, (16,), 0)

NPC = NP // 2
NPA = NPC + 16
TILE_A = EP // 32
TILE_B = EP // 16


def _scale_rows_group(rows, ridx, ev):
    for k in range(128):
        cidx = jnp.full((16,), k, jnp.int32)
        v = plsc.load_gather(rows, [ridx, cidx])
        plsc.store_scatter(rows, [ridx, cidx], v * ev)


def _gate_alpha_body(src_h, dst_h, ea_h, u_h, ra_h, wcol_h, attl_h, e_h,
                     srcv, dstv, eav, rows, evec, rv, wcolv, attlv, sem):
    c = lax.axis_index("c")
    s = lax.axis_index("s")
    wid = s * 2 + c
    pltpu.sync_copy(ra_h, rv)
    pltpu.sync_copy(wcol_h, wcolv)
    pltpu.sync_copy(attl_h, attlv)
    ebase = wid * TILE_A

    def chunk(i, _):
        base = ebase + i * EC
        pltpu.sync_copy(src_h.at[pl.ds(base, EC)], srcv)
        pltpu.sync_copy(dst_h.at[pl.ds(base, EC)], dstv)
        pltpu.sync_copy(ea_h.at[pl.ds(base, EC)], eav)
        pltpu.async_copy(u_h.at[srcv], rows, sem).wait()

        def group(g, _):
            sl = pl.ds(g * 16, 16)
            di = dstv[sl]
            eag = eav[sl]
            rg = plsc.load_gather(rv, [di])
            ridx = g * 16 + _iota16()

            dot = jnp.zeros((16,), jnp.float32)
            for k in range(128):
                ksplat = jnp.full((16,), k, jnp.int32)
                wk = plsc.load_gather(wcolv, [ksplat])
                alk = plsc.load_gather(attlv, [ksplat])
                ucol = plsc.load_gather(rows, [ridx, ksplat])
                he = _leaky(ucol + eag * wk)
                dot = dot + he * alk
            a = _leaky(dot + rg)
            gid = base + g * 16 + _iota16()
            evec[sl] = jnp.where(gid < N_EDGES, jnp.exp(a), 0.0)
            return 0
        lax.fori_loop(0, EC // 16, group, 0)
        pltpu.sync_copy(evec, e_h.at[pl.ds(base, EC)])
        return 0
    lax.fori_loop(0, TILE_A // EC, chunk, 0)


def _gate_alpha_call(src_p, dst_p, ea_p, u, ra, wcol, attl):
    mesh = plsc.VectorSubcoreMesh(core_axis_name="c", subcore_axis_name="s")
    k = pl.kernel(
        _gate_alpha_body, mesh=mesh,
        out_type=jax.ShapeDtypeStruct((EP,), jnp.float32),
        compiler_params=pltpu.CompilerParams(needs_layout_passes=False),
        scratch_types=[
            pltpu.VMEM((EC,), jnp.int32),
            pltpu.VMEM((EC,), jnp.int32),
            pltpu.VMEM((EC,), jnp.float32),
            pltpu.VMEM((EC, 128), jnp.float32),
            pltpu.VMEM((EC,), jnp.float32),
            pltpu.VMEM((NP,), jnp.float32),
            pltpu.VMEM((128,), jnp.float32),
            pltpu.VMEM((128,), jnp.float32),
            pltpu.SemaphoreType.DMA,
        ])
    return k(src_p, dst_p, ea_p, u, ra, wcol, attl)


CAP = 6144
CAP16 = CAP + 16
EPB = 32 * CAP
ECB = 512
TILE_S = EPB // 16
NCH = TILE_S // ECB


def _part_body(src_h, dst_h, src2_h, dst2_h, eid2_h,
               srcv, dstv, osrc0, odst0, oeid0, osrc1, odst1, oeid1):
    c = lax.axis_index("c")
    s = lax.axis_index("s")
    wid = s * 2 + c

    def fill(j, _):
        sl = pl.ds(j * 16, 16)
        z = jnp.zeros((16,), jnp.int32)
        npv = jnp.full((16,), NP, jnp.int32)
        osrc0[sl] = z
        odst0[sl] = npv
        oeid0[sl] = z
        osrc1[sl] = z
        odst1[sl] = npv
        oeid1[sl] = z
        return 0
    lax.fori_loop(0, CAP16 // 16, fill, 0)

    ebase = wid * TILE_E

    def chunk(i, carry):
        base = ebase + i * EC
        pltpu.sync_copy(src_h.at[pl.ds(base, EC)], srcv)
        pltpu.sync_copy(dst_h.at[pl.ds(base, EC)], dstv)

        def group(g, pp):
            p0, p1 = pp
            sl = pl.ds(g * 16, 16)
            sv16 = srcv[sl]
            dv16 = dstv[sl]
            eidv = base + g * 16 + _iota16()
            real = eidv < N_EDGES
            m0 = real & (dv16 < NPC)
            m1 = real & (dv16 >= NPC)
            cs0 = plsc.cumsum(m0.astype(jnp.int32))
            pos0 = p0 + cs0 - m0.astype(jnp.int32)
            plsc.store_scatter(osrc0, [pos0], sv16, mask=m0)
            plsc.store_scatter(odst0, [pos0], dv16, mask=m0)
            plsc.store_scatter(oeid0, [pos0], eidv, mask=m0)
            cs1 = plsc.cumsum(m1.astype(jnp.int32))
            pos1 = p1 + cs1 - m1.astype(jnp.int32)
            plsc.store_scatter(osrc1, [pos1], sv16, mask=m1)
            plsc.store_scatter(odst1, [pos1], dv16, mask=m1)
            plsc.store_scatter(oeid1, [pos1], eidv, mask=m1)
            return (jnp.minimum(p0 + jnp.max(cs0), CAP),
                    jnp.minimum(p1 + jnp.max(cs1), CAP))
        return lax.fori_loop(0, EC // 16, group, carry)
    lax.fori_loop(0, TILE_E // EC, chunk,
                  (jnp.int32(0), jnp.int32(0)))

    b0 = wid * CAP
    b1 = EPB + wid * CAP
    pltpu.sync_copy(osrc0.at[pl.ds(0, CAP)], src2_h.at[pl.ds(b0, CAP)])
    pltpu.sync_copy(odst0.at[pl.ds(0, CAP)], dst2_h.at[pl.ds(b0, CAP)])
    pltpu.sync_copy(oeid0.at[pl.ds(0, CAP)], eid2_h.at[pl.ds(b0, CAP)])
    pltpu.sync_copy(osrc1.at[pl.ds(0, CAP)], src2_h.at[pl.ds(b1, CAP)])
    pltpu.sync_copy(odst1.at[pl.ds(0, CAP)], dst2_h.at[pl.ds(b1, CAP)])
    pltpu.sync_copy(oeid1.at[pl.ds(0, CAP)], eid2_h.at[pl.ds(b1, CAP)])


def _part_call(src_p, dst_p):
    mesh = plsc.VectorSubcoreMesh(core_axis_name="c", subcore_axis_name="s")
    k = pl.kernel(
        _part_body, mesh=mesh,
        out_type=[jax.ShapeDtypeStruct((2 * EPB,), jnp.int32)] * 3,
        compiler_params=pltpu.CompilerParams(needs_layout_passes=False),
        scratch_types=(
            [pltpu.VMEM((EC,), jnp.int32)] * 2
            + [pltpu.VMEM((CAP16,), jnp.int32)] * 6
        ))
    return k(src_p, dst_p)


def _scatter_body(is_conv, src_h, dst_h, eid_h, w_h, wb_h, table_h,
                  z2_h, z1_h, acc_h, den_h,
                  srcv, dstv, eidv, dstl, rows, evec, sv, dv,
                  acc_sh, den_sh, sem):
    c = lax.axis_index("c")
    s = lax.axis_index("s")
    if is_conv:
        pltpu.sync_copy(w_h, sv.at[pl.ds(0, NP)])
        pltpu.sync_copy(wb_h, dv.at[pl.ds(0, NP)])

    @pl.when(s == 0)
    def _():
        pltpu.sync_copy(z2_h, acc_sh)
        pltpu.sync_copy(z1_h, den_sh)
    plsc.subcore_barrier()

    lo = c * NPC

    def chunk(i, _):
        base = c * EPB + s * TILE_S + i * ECB
        pltpu.sync_copy(src_h.at[pl.ds(base, ECB)], srcv)
        pltpu.sync_copy(dst_h.at[pl.ds(base, ECB)], dstv)
        if not is_conv:
            pltpu.sync_copy(eid_h.at[pl.ds(base, ECB)], eidv)
            pltpu.async_copy(w_h.at[eidv], evec, sem).wait()
        pltpu.async_copy(table_h.at[srcv], rows, sem).wait()

        def group(g, _):
            sl = pl.ds(g * 16, 16)
            di = dstv[sl]
            if is_conv:
                si = srcv[sl]
                a = _leaky(plsc.load_gather(sv, [si])
                           + plsc.load_gather(dv, [di]))
                ev = jnp.where(di < NP, jnp.exp(a), 0.0)
                evec[sl] = ev
            else:
                ev = evec[sl]
            local = di - lo
            ok = (di >= lo) & (local < NPC)
            dstl[sl] = jnp.where(ok, local, NPC)
            ridx = g * 16 + _iota16()
            for k in range(128):
                cidx = jnp.full((16,), k, jnp.int32)
                v = plsc.load_gather(rows, [ridx, cidx])
                plsc.store_scatter(rows, [ridx, cidx], v * ev)
            return 0
        lax.fori_loop(0, ECB // 16, group, 0)
        pltpu.sync_copy(rows, acc_sh.at[dstl], add=True)
        pltpu.sync_copy(evec, den_sh.at[dstl], add=True)
        return 0
    lax.fori_loop(0, NCH, chunk, 0)

    plsc.subcore_barrier()
    rpt = NPC // 16
    r0 = s * rpt
    pltpu.sync_copy(acc_sh.at[pl.ds(r0, rpt)], rows.at[pl.ds(0, rpt)])
    pltpu.sync_copy(rows.at[pl.ds(0, rpt)], acc_h.at[c, pl.ds(r0, rpt)])
    pltpu.sync_copy(den_sh.at[pl.ds(r0, rpt)], evec.at[pl.ds(0, rpt)])
    pltpu.sync_copy(evec.at[pl.ds(0, rpt)],
                    den_h.at[pl.ds(c * NPC + r0, rpt)])


def _scatter_call(is_conv, src2, dst2, eid2, w, wb, table):
    mesh = plsc.VectorSubcoreMesh(core_axis_name="c", subcore_axis_name="s")
    k = pl.kernel(
        functools.partial(_scatter_body, is_conv), mesh=mesh,
        out_type=[jax.ShapeDtypeStruct((2, NPC, 128), jnp.float32),
                  jax.ShapeDtypeStruct((2 * NPC,), jnp.float32)],
        compiler_params=pltpu.CompilerParams(needs_layout_passes=False),
        scratch_types=[
            pltpu.VMEM((ECB,), jnp.int32),
            pltpu.VMEM((ECB,), jnp.int32),
            pltpu.VMEM((ECB,), jnp.int32),
            pltpu.VMEM((ECB,), jnp.int32),
            pltpu.VMEM((ECB, 128), jnp.float32),
            pltpu.VMEM((ECB,), jnp.float32),
            pltpu.VMEM((NP + 16,), jnp.float32),
            pltpu.VMEM((NP + 16,), jnp.float32),
            pltpu.VMEM_SHARED((NPA, 128), jnp.float32),
            pltpu.VMEM_SHARED((NPA,), jnp.float32),
            pltpu.SemaphoreType.DMA,
        ])
    z2 = jnp.zeros((NPA, 128), jnp.float32)
    z1 = jnp.zeros((NPA,), jnp.float32)
    acc, den = k(src2, dst2, eid2, w, wb, table, z2, z1)
    return acc.reshape(NP, 128), den.reshape(NP, 1)


def _pre_body(x_ref, wlin1t_ref, blin1_ref, u1t_ref, g2t_ref, attr_ref,
              x1_ref, u_ref, y_ref, r_ref):
    xb = x_ref[...]
    x1 = _leaky(jnp.dot(xb, wlin1t_ref[...],
                        preferred_element_type=jnp.float32) + blin1_ref[...])
    x1_ref[...] = x1
    u_ref[...] = jnp.dot(x1, u1t_ref[...], preferred_element_type=jnp.float32)
    y_ref[...] = jnp.dot(x1, g2t_ref[...], preferred_element_type=jnp.float32)
    r_ref[...] = jnp.dot(x1, attr_ref[...], preferred_element_type=jnp.float32)


def _pre_call(x_p, wlin1t, blin1, u1t, g2t, attr):
    n_blk = NP // BLK
    row = pl.BlockSpec((BLK, 128), lambda i: (i, 0))
    full = lambda shape: pl.BlockSpec(shape, lambda i: (0, 0))
    return pl.pallas_call(
        _pre_body,
        grid=(n_blk,),
        in_specs=[row, full((128, 128)), full((1, 128)), full((128, 128)),
                  full((128, 128)), full((128, 1))],
        out_specs=[row, row, row, pl.BlockSpec((BLK, 1), lambda i: (i, 0))],
        out_shape=[jax.ShapeDtypeStruct((NP, 128), jnp.float32)] * 3
        + [jax.ShapeDtypeStruct((NP, 1), jnp.float32)],
    )(x_p, wlin1t, blin1, u1t, g2t, attr)


def _gru_body(has_next, num_ref, den_ref, bias_ref, x_ref,
              wiht_ref, whht_ref, bih_ref, bhh_ref, *rest):
    if has_next:
        (convt_ref, asrc_ref, adst_ref,
         xn_ref, xt_ref, s_ref, d_ref) = rest
    else:
        (xn_ref,) = rest
    num = num_ref[...]
    den = den_ref[...]
    h = num / (den + EPS) + bias_ref[...]
    h = jnp.where(h > 0, h, jnp.exp(jnp.minimum(h, 0.0)) - 1.0)
    x = x_ref[...]
    gi = jnp.dot(h, wiht_ref[...], preferred_element_type=jnp.float32) + bih_ref[...]
    gh = jnp.dot(x, whht_ref[...], preferred_element_type=jnp.float32) + bhh_ref[...]
    r = jax.nn.sigmoid(gi[:, :128] + gh[:, :128])
    z = jax.nn.sigmoid(gi[:, 128:256] + gh[:, 128:256])
    n = jnp.tanh(gi[:, 256:] + r * gh[:, 256:])
    xn = jnp.maximum((1.0 - z) * n + z * x, 0.0)
    xn_ref[...] = xn
    if has_next:
        xt = jnp.dot(xn, convt_ref[...], preferred_element_type=jnp.float32)
        xt_ref[...] = xt
        s_ref[...] = jnp.dot(xt, asrc_ref[...], preferred_element_type=jnp.float32)
        d_ref[...] = jnp.dot(xt, adst_ref[...], preferred_element_type=jnp.float32)


def _gru_call(num, den, bias, x, wiht, whht, bih, bhh, nxt=None):
    n_blk = NP // BLK
    row = pl.BlockSpec((BLK, 128), lambda i: (i, 0))
    col = pl.BlockSpec((BLK, 1), lambda i: (i, 0))
    full = lambda shape: pl.BlockSpec(shape, lambda i: tuple(0 for _ in shape))
    in_specs = [row, col, full((1, 128)), row, full((128, 384)),
                full((128, 384)), full((1, 384)), full((1, 384))]
    args = [num, den, bias, x, wiht, whht, bih, bhh]
    if nxt is None:
        out_specs = [row]
        out_shape = [jax.ShapeDtypeStruct((NP, 128), jnp.float32)]
    else:
        convt, asrc, adst = nxt
        in_specs += [full((128, 128)), full((128, 1)), full((128, 1))]
        args += [convt, asrc, adst]
        out_specs = [row, row, col, col]
        out_shape = [jax.ShapeDtypeStruct((NP, 128), jnp.float32),
                     jax.ShapeDtypeStruct((NP, 128), jnp.float32),
                     jax.ShapeDtypeStruct((NP, 1), jnp.float32),
                     jax.ShapeDtypeStruct((NP, 1), jnp.float32)]
    res = pl.pallas_call(
        functools.partial(_gru_body, nxt is not None),
        grid=(n_blk,),
        in_specs=in_specs, out_specs=out_specs, out_shape=out_shape,
    )(*args)
    return res


def _read_body(x_ref, batch_row_ref, batch_col_ref, srct_ref, dstt_ref,
               asrc_ref, adst_ref, mbias_ref, wiht_ref, whht_ref,
               bih_ref, bhh_ref, wlin2t_ref, blin2_ref, wfc1t_ref, bfc1_ref,
               out_ref):
    x = x_ref[...]
    batch_row = batch_row_ref[...]
    batch_col = batch_col_ref[...]
    gids0 = jax.lax.broadcasted_iota(jnp.int32, (NUM_GRAPHS, NP), 0)
    smask = (gids0 == batch_row).astype(jnp.float32)
    gids1 = jax.lax.broadcasted_iota(jnp.int32, (NP, NUM_GRAPHS), 1)
    tmask = (gids1 == batch_col).astype(jnp.float32)

    out = jnp.maximum(
        jnp.dot(smask, x, preferred_element_type=jnp.float32), 0.0)
    xs = jnp.dot(x, srct_ref[...], preferred_element_type=jnp.float32)
    sa = jnp.dot(xs, asrc_ref[...], preferred_element_type=jnp.float32)

    wiht = wiht_ref[...]
    whht = whht_ref[...]
    bih = bih_ref[...]
    bhh = bhh_ref[...]
    mbias = mbias_ref[...]
    dstt = dstt_ref[...]
    adst = adst_ref[...]

    def step(_, out):
        xd = jnp.dot(out, dstt, preferred_element_type=jnp.float32)
        db = jnp.dot(xd, adst, preferred_element_type=jnp.float32)
        apn = jnp.dot(tmask, db, preferred_element_type=jnp.float32)
        a = _leaky(sa + apn)
        e = jnp.exp(a)
        num = jnp.dot(smask, xs * e, preferred_element_type=jnp.float32)
        den = jnp.dot(smask, e, preferred_element_type=jnp.float32)
        h = num / (den + EPS) + mbias
        h = jnp.where(h > 0, h, jnp.exp(jnp.minimum(h, 0.0)) - 1.0)
        gi = jnp.dot(h, wiht, preferred_element_type=jnp.float32) + bih
        gh = jnp.dot(out, whht, preferred_element_type=jnp.float32) + bhh
        r = jax.nn.sigmoid(gi[:, :128] + gh[:, :128])
        z = jax.nn.sigmoid(gi[:, 128:256] + gh[:, 128:256])
        n = jnp.tanh(gi[:, 256:] + r * gh[:, 256:])
        return jnp.maximum((1.0 - z) * n + z * out, 0.0)

    out = jax.lax.fori_loop(0, NUM_TIMESTEPS, step, out)
    out = jnp.maximum(
        jnp.dot(out, wlin2t_ref[...], preferred_element_type=jnp.float32)
        + blin2_ref[...], 0.0)
    out_ref[...] = (jnp.dot(out, wfc1t_ref[...], preferred_element_type=jnp.float32)
                    + bfc1_ref[...])


def _read_call(x, batch_row, batch_col, srct, dstt, asrc, adst, mbias,
               wiht, whht, bih, bhh, wlin2t, blin2, wfc1t, bfc1):
    full = lambda shape: pl.BlockSpec(shape, lambda: (0,) * len(shape))
    return pl.pallas_call(
        _read_body,
        in_specs=[full((NP, 128)), full((1, NP)), full((NP, 1)),
                  full((128, 128)), full((128, 128)), full((128, 1)),
                  full((128, 1)), full((1, 128)), full((128, 384)),
                  full((128, 384)), full((1, 384)), full((1, 384)),
                  full((128, 128)), full((1, 128)), full((128, 128)),
                  full((1, 128))],
        out_specs=full((NUM_GRAPHS, 128)),
        out_shape=jax.ShapeDtypeStruct((NUM_GRAPHS, 128), jnp.float32),
    )(x, batch_row, batch_col, srct, dstt, asrc, adst, mbias,
      wiht, whht, bih, bhh, wlin2t, blin2, wfc1t, bfc1)


def kernel(x, edge_index, edge_attr, batch,
           w_lin1, b_lin1,
           gate_lin1_w, gate_att_l, gate_att_r, gate_lin2_w, gate_bias,
           gru0_wih, gru0_whh, gru0_bih, gru0_bhh,
           gru1_wih, gru1_whh, gru1_bih, gru1_bhh,
           gru2_wih, gru2_whh, gru2_bih, gru2_bhh,
           gru3_wih, gru3_whh, gru3_bih, gru3_bhh,
           conv1_w, conv1_att_src, conv1_att_dst, conv1_bias,
           conv2_w, conv2_att_src, conv2_att_dst, conv2_bias,
           conv3_w, conv3_att_src, conv3_att_dst, conv3_bias,
           mol_lin_src_w, mol_lin_dst_w, mol_att_src, mol_att_dst, mol_bias,
           molgru_wih, molgru_whh, molgru_bih, molgru_bhh,
           w_lin2, b_lin2, w_fc1, b_fc1):
    f32 = jnp.float32
    x_p = jnp.pad(x, ((0, NP - N_NODES), (0, 0)))
    src_p = jnp.pad(edge_index[0], (0, EP - N_EDGES))
    dst_p = jnp.pad(edge_index[1], (0, EP - N_EDGES))
    ea_p = jnp.pad(edge_attr[:, 0], (0, EP - N_EDGES))

    u1t = gate_lin1_w[:, :128].T
    wcol = gate_lin1_w[:, 128]
    x1, u, y, r = _pre_call(x_p, w_lin1.T, b_lin1[None, :], u1t,
                            gate_lin2_w.T, gate_att_r[:, None])

    src2, dst2, eid2 = _part_call(src_p, dst_p)

    e_p = _gate_alpha_call(src_p, dst_p, ea_p, u, r.reshape(NP),
                           wcol, gate_att_l)
    num, den = _scatter_call(False, src2, dst2, eid2, e_p, e_p, y)

    convs = [(conv1_w, conv1_att_src, conv1_att_dst, conv1_bias),
             (conv2_w, conv2_att_src, conv2_att_dst, conv2_bias),
             (conv3_w, conv3_att_src, conv3_att_dst, conv3_bias)]
    grus = [(gru0_wih, gru0_whh, gru0_bih, gru0_bhh),
            (gru1_wih, gru1_whh, gru1_bih, gru1_bhh),
            (gru2_wih, gru2_whh, gru2_bih, gru2_bhh),
            (gru3_wih, gru3_whh, gru3_bih, gru3_bhh)]

    x_cur = x1
    bias_cur = gate_bias
    for l in range(NUM_LAYERS):
        wih, whh, bih, bhh = grus[l]
        if l < NUM_LAYERS - 1:
            cw, ca_s, ca_d, cb = convs[l]
            x_cur, xt, s, d = _gru_call(
                num, den, bias_cur[None, :], x_cur, wih.T, whh.T,
                bih[None, :], bhh[None, :],
                nxt=(cw.T, ca_s[:, None], ca_d[:, None]))
            num, den = _scatter_call(True, src2, dst2, eid2,
                                     s.reshape(NP), d.reshape(NP), xt)
            bias_cur = cb
        else:
            (x_cur,) = _gru_call(num, den, bias_cur[None, :], x_cur,
                                 wih.T, whh.T, bih[None, :], bhh[None, :])

    batch_p = jnp.full((NP,), NUM_GRAPHS, jnp.int32).at[:N_NODES].set(batch)
    out = _read_call(
        x_cur, batch_p[None, :], batch_p[:, None],
        mol_lin_src_w.T, mol_lin_dst_w.T, mol_att_src[:, None],
        mol_att_dst[:, None], mol_bias[None, :],
        molgru_wih.T, molgru_whh.T, molgru_bih[None, :], molgru_bhh[None, :],
        w_lin2.T, b_lin2[None, :], w_fc1.T, b_fc1[None, :])
    return out


# speedup vs baseline: 1.1021x; 1.1021x over previous
import functools

import jax
import jax.numpy as jnp
from jax import lax
from jax.experimental import pallas as pl
from jax.experimental.pallas import tpu as pltpu
from jax.experimental.pallas import tpu_sc as plsc

N_NODES = 10000
N_EDGES = 320000
HID = 128
NUM_GRAPHS = 64
NUM_LAYERS = 4
NUM_TIMESTEPS = 16

NP = 10240
BLK = 1024
EPS = 1e-16

NW = 32
EP = 327680
TILE_E = EP // NW
EC = 512
ROWS_T = NP // 16


def _leaky(v):
    return jnp.where(v > 0, v, 0.01 * v)


_iota16 = lambda: lax.broadcasted_iota(jnp.int32, (16,), 0)

NPC = NP // 2
NPA = NPC + 16
TILE_A = EP // 32
TILE_B = EP // 16


def _scale_rows_group(rows, ridx, ev):
    for k in range(128):
        cidx = jnp.full((16,), k, jnp.int32)
        v = plsc.load_gather(rows, [ridx, cidx])
        plsc.store_scatter(rows, [ridx, cidx], v * ev)


def _gate_alpha_body(src_h, dst_h, ea_h, u_h, ra_h, wcol_h, attl_h, e_h,
                     srcv, dstv, eav, rows, evec, rv, wcolv, attlv, sem):
    c = lax.axis_index("c")
    s = lax.axis_index("s")
    wid = s * 2 + c
    pltpu.sync_copy(ra_h, rv)
    pltpu.sync_copy(wcol_h, wcolv)
    pltpu.sync_copy(attl_h, attlv)
    ebase = wid * TILE_A

    def chunk(i, _):
        base = ebase + i * EC
        pltpu.sync_copy(src_h.at[pl.ds(base, EC)], srcv)
        pltpu.sync_copy(dst_h.at[pl.ds(base, EC)], dstv)
        pltpu.sync_copy(ea_h.at[pl.ds(base, EC)], eav)
        pltpu.async_copy(u_h.at[srcv], rows, sem).wait()

        def group(g, _):
            sl = pl.ds(g * 16, 16)
            di = dstv[sl]
            eag = eav[sl]
            rg = plsc.load_gather(rv, [di])
            ridx = g * 16 + _iota16()

            dot = jnp.zeros((16,), jnp.float32)
            for k in range(128):
                ksplat = jnp.full((16,), k, jnp.int32)
                wk = plsc.load_gather(wcolv, [ksplat])
                alk = plsc.load_gather(attlv, [ksplat])
                ucol = plsc.load_gather(rows, [ridx, ksplat])
                he = _leaky(ucol + eag * wk)
                dot = dot + he * alk
            a = _leaky(dot + rg)
            gid = base + g * 16 + _iota16()
            evec[sl] = jnp.where(gid < N_EDGES, jnp.exp(a), 0.0)
            return 0
        lax.fori_loop(0, EC // 16, group, 0)
        pltpu.sync_copy(evec, e_h.at[pl.ds(base, EC)])
        return 0
    lax.fori_loop(0, TILE_A // EC, chunk, 0)


def _gate_alpha_call(src_p, dst_p, ea_p, u, ra, wcol, attl):
    mesh = plsc.VectorSubcoreMesh(core_axis_name="c", subcore_axis_name="s")
    k = pl.kernel(
        _gate_alpha_body, mesh=mesh,
        out_type=jax.ShapeDtypeStruct((EP,), jnp.float32),
        compiler_params=pltpu.CompilerParams(needs_layout_passes=False),
        scratch_types=[
            pltpu.VMEM((EC,), jnp.int32),
            pltpu.VMEM((EC,), jnp.int32),
            pltpu.VMEM((EC,), jnp.float32),
            pltpu.VMEM((EC, 128), jnp.float32),
            pltpu.VMEM((EC,), jnp.float32),
            pltpu.VMEM((NP,), jnp.float32),
            pltpu.VMEM((128,), jnp.float32),
            pltpu.VMEM((128,), jnp.float32),
            pltpu.SemaphoreType.DMA,
        ])
    return k(src_p, dst_p, ea_p, u, ra, wcol, attl)


ECB = 512
NCH = TILE_B // ECB


def _scatter_body(is_conv, src_h, dst_h, w_h, wb_h, table_h, z2_h, z1_h,
                  acc_h, den_h,
                  srcv, dstv, dstl, rows, evec, sv, dv, acc_sh, den_sh, sem):
    c = lax.axis_index("c")
    s = lax.axis_index("s")
    if is_conv:
        pltpu.sync_copy(w_h, sv)
        pltpu.sync_copy(wb_h, dv)

    @pl.when(s == 0)
    def _():
        pltpu.sync_copy(z2_h, acc_sh)
        pltpu.sync_copy(z1_h, den_sh)
    plsc.subcore_barrier()

    lo = c * NPC

    def chunk(i, _):
        base = s * TILE_B + i * ECB
        pltpu.sync_copy(src_h.at[pl.ds(base, ECB)], srcv)
        pltpu.sync_copy(dst_h.at[pl.ds(base, ECB)], dstv)
        if not is_conv:
            pltpu.sync_copy(w_h.at[pl.ds(base, ECB)], evec)
        pltpu.async_copy(table_h.at[srcv], rows, sem).wait()

        def group(g, _):
            sl = pl.ds(g * 16, 16)
            di = dstv[sl]
            if is_conv:
                si = srcv[sl]
                a = _leaky(plsc.load_gather(sv, [si])
                           + plsc.load_gather(dv, [di]))
                gid = base + g * 16 + _iota16()
                ev = jnp.where(gid < N_EDGES, jnp.exp(a), 0.0)
                evec[sl] = ev
            else:
                ev = evec[sl]
            local = di - lo
            ok = (di >= lo) & (local < NPC)
            dstl[sl] = jnp.where(ok, local, NPC)
            ridx = g * 16 + _iota16()
            for k in range(128):
                cidx = jnp.full((16,), k, jnp.int32)
                v = plsc.load_gather(rows, [ridx, cidx])
                plsc.store_scatter(rows, [ridx, cidx], v * ev)
            return 0
        lax.fori_loop(0, ECB // 16, group, 0)
        pltpu.sync_copy(rows, acc_sh.at[dstl], add=True)
        pltpu.sync_copy(evec, den_sh.at[dstl], add=True)
        return 0
    lax.fori_loop(0, NCH, chunk, 0)

    plsc.subcore_barrier()
    rpt = NPC // 16
    r0 = s * rpt
    pltpu.sync_copy(acc_sh.at[pl.ds(r0, rpt)], rows.at[pl.ds(0, rpt)])
    pltpu.sync_copy(rows.at[pl.ds(0, rpt)], acc_h.at[c, pl.ds(r0, rpt)])
    pltpu.sync_copy(den_sh.at[pl.ds(r0, rpt)], evec.at[pl.ds(0, rpt)])
    pltpu.sync_copy(evec.at[pl.ds(0, rpt)],
                    den_h.at[pl.ds(c * NPC + r0, rpt)])


def _scatter_call(is_conv, src_p, dst_p, w, wb, table):
    mesh = plsc.VectorSubcoreMesh(core_axis_name="c", subcore_axis_name="s")
    k = pl.kernel(
        functools.partial(_scatter_body, is_conv), mesh=mesh,
        out_type=[jax.ShapeDtypeStruct((2, NPC, 128), jnp.float32),
                  jax.ShapeDtypeStruct((2 * NPC,), jnp.float32)],
        compiler_params=pltpu.CompilerParams(needs_layout_passes=False),
        scratch_types=[
            pltpu.VMEM((ECB,), jnp.int32),
            pltpu.VMEM((ECB,), jnp.int32),
            pltpu.VMEM((ECB,), jnp.int32),
            pltpu.VMEM((ECB, 128), jnp.float32),
            pltpu.VMEM((ECB,), jnp.float32),
            pltpu.VMEM((NP,), jnp.float32),
            pltpu.VMEM((NP,), jnp.float32),
            pltpu.VMEM_SHARED((NPA, 128), jnp.float32),
            pltpu.VMEM_SHARED((NPA,), jnp.float32),
            pltpu.SemaphoreType.DMA,
        ])
    z2 = jnp.zeros((NPA, 128), jnp.float32)
    z1 = jnp.zeros((NPA,), jnp.float32)
    acc, den = k(src_p, dst_p, w, wb, table, z2, z1)
    return acc.reshape(NP, 128), den.reshape(NP, 1)


def _pre_body(x_ref, wlin1t_ref, blin1_ref, u1t_ref, g2t_ref, attr_ref,
              x1_ref, u_ref, y_ref, r_ref):
    xb = x_ref[...]
    x1 = _leaky(jnp.dot(xb, wlin1t_ref[...],
                        preferred_element_type=jnp.float32) + blin1_ref[...])
    x1_ref[...] = x1
    u_ref[...] = jnp.dot(x1, u1t_ref[...], preferred_element_type=jnp.float32)
    y_ref[...] = jnp.dot(x1, g2t_ref[...], preferred_element_type=jnp.float32)
    r_ref[...] = jnp.dot(x1, attr_ref[...], preferred_element_type=jnp.float32)


def _pre_call(x_p, wlin1t, blin1, u1t, g2t, attr):
    n_blk = NP // BLK
    row = pl.BlockSpec((BLK, 128), lambda i: (i, 0))
    full = lambda shape: pl.BlockSpec(shape, lambda i: (0, 0))
    return pl.pallas_call(
        _pre_body,
        grid=(n_blk,),
        in_specs=[row, full((128, 128)), full((1, 128)), full((128, 128)),
                  full((128, 128)), full((128, 1))],
        out_specs=[row, row, row, pl.BlockSpec((BLK, 1), lambda i: (i, 0))],
        out_shape=[jax.ShapeDtypeStruct((NP, 128), jnp.float32)] * 3
        + [jax.ShapeDtypeStruct((NP, 1), jnp.float32)],
    )(x_p, wlin1t, blin1, u1t, g2t, attr)


def _gru_body(has_next, num_ref, den_ref, bias_ref, x_ref,
              wiht_ref, whht_ref, bih_ref, bhh_ref, *rest):
    if has_next:
        (convt_ref, asrc_ref, adst_ref,
         xn_ref, xt_ref, s_ref, d_ref) = rest
    else:
        (xn_ref,) = rest
    num = num_ref[...]
    den = den_ref[...]
    h = num / (den + EPS) + bias_ref[...]
    h = jnp.where(h > 0, h, jnp.exp(jnp.minimum(h, 0.0)) - 1.0)
    x = x_ref[...]
    gi = jnp.dot(h, wiht_ref[...], preferred_element_type=jnp.float32) + bih_ref[...]
    gh = jnp.dot(x, whht_ref[...], preferred_element_type=jnp.float32) + bhh_ref[...]
    r = jax.nn.sigmoid(gi[:, :128] + gh[:, :128])
    z = jax.nn.sigmoid(gi[:, 128:256] + gh[:, 128:256])
    n = jnp.tanh(gi[:, 256:] + r * gh[:, 256:])
    xn = jnp.maximum((1.0 - z) * n + z * x, 0.0)
    xn_ref[...] = xn
    if has_next:
        xt = jnp.dot(xn, convt_ref[...], preferred_element_type=jnp.float32)
        xt_ref[...] = xt
        s_ref[...] = jnp.dot(xt, asrc_ref[...], preferred_element_type=jnp.float32)
        d_ref[...] = jnp.dot(xt, adst_ref[...], preferred_element_type=jnp.float32)


def _gru_call(num, den, bias, x, wiht, whht, bih, bhh, nxt=None):
    n_blk = NP // BLK
    row = pl.BlockSpec((BLK, 128), lambda i: (i, 0))
    col = pl.BlockSpec((BLK, 1), lambda i: (i, 0))
    full = lambda shape: pl.BlockSpec(shape, lambda i: tuple(0 for _ in shape))
    in_specs = [row, col, full((1, 128)), row, full((128, 384)),
                full((128, 384)), full((1, 384)), full((1, 384))]
    args = [num, den, bias, x, wiht, whht, bih, bhh]
    if nxt is None:
        out_specs = [row]
        out_shape = [jax.ShapeDtypeStruct((NP, 128), jnp.float32)]
    else:
        convt, asrc, adst = nxt
        in_specs += [full((128, 128)), full((128, 1)), full((128, 1))]
        args += [convt, asrc, adst]
        out_specs = [row, row, col, col]
        out_shape = [jax.ShapeDtypeStruct((NP, 128), jnp.float32),
                     jax.ShapeDtypeStruct((NP, 128), jnp.float32),
                     jax.ShapeDtypeStruct((NP, 1), jnp.float32),
                     jax.ShapeDtypeStruct((NP, 1), jnp.float32)]
    res = pl.pallas_call(
        functools.partial(_gru_body, nxt is not None),
        grid=(n_blk,),
        in_specs=in_specs, out_specs=out_specs, out_shape=out_shape,
    )(*args)
    return res


def _read_body(x_ref, batch_row_ref, batch_col_ref, srct_ref, dstt_ref,
               asrc_ref, adst_ref, mbias_ref, wiht_ref, whht_ref,
               bih_ref, bhh_ref, wlin2t_ref, blin2_ref, wfc1t_ref, bfc1_ref,
               out_ref):
    x = x_ref[...]
    batch_row = batch_row_ref[...]
    batch_col = batch_col_ref[...]
    gids0 = jax.lax.broadcasted_iota(jnp.int32, (NUM_GRAPHS, NP), 0)
    smask = (gids0 == batch_row).astype(jnp.float32)
    gids1 = jax.lax.broadcasted_iota(jnp.int32, (NP, NUM_GRAPHS), 1)
    tmask = (gids1 == batch_col).astype(jnp.float32)

    out = jnp.maximum(
        jnp.dot(smask, x, preferred_element_type=jnp.float32), 0.0)
    xs = jnp.dot(x, srct_ref[...], preferred_element_type=jnp.float32)
    sa = jnp.dot(xs, asrc_ref[...], preferred_element_type=jnp.float32)

    wiht = wiht_ref[...]
    whht = whht_ref[...]
    bih = bih_ref[...]
    bhh = bhh_ref[...]
    mbias = mbias_ref[...]
    dstt = dstt_ref[...]
    adst = adst_ref[...]

    def step(_, out):
        xd = jnp.dot(out, dstt, preferred_element_type=jnp.float32)
        db = jnp.dot(xd, adst, preferred_element_type=jnp.float32)
        apn = jnp.dot(tmask, db, preferred_element_type=jnp.float32)
        a = _leaky(sa + apn)
        e = jnp.exp(a)
        num = jnp.dot(smask, xs * e, preferred_element_type=jnp.float32)
        den = jnp.dot(smask, e, preferred_element_type=jnp.float32)
        h = num / (den + EPS) + mbias
        h = jnp.where(h > 0, h, jnp.exp(jnp.minimum(h, 0.0)) - 1.0)
        gi = jnp.dot(h, wiht, preferred_element_type=jnp.float32) + bih
        gh = jnp.dot(out, whht, preferred_element_type=jnp.float32) + bhh
        r = jax.nn.sigmoid(gi[:, :128] + gh[:, :128])
        z = jax.nn.sigmoid(gi[:, 128:256] + gh[:, 128:256])
        n = jnp.tanh(gi[:, 256:] + r * gh[:, 256:])
        return jnp.maximum((1.0 - z) * n + z * out, 0.0)

    out = jax.lax.fori_loop(0, NUM_TIMESTEPS, step, out)
    out = jnp.maximum(
        jnp.dot(out, wlin2t_ref[...], preferred_element_type=jnp.float32)
        + blin2_ref[...], 0.0)
    out_ref[...] = (jnp.dot(out, wfc1t_ref[...], preferred_element_type=jnp.float32)
                    + bfc1_ref[...])


def _read_call(x, batch_row, batch_col, srct, dstt, asrc, adst, mbias,
               wiht, whht, bih, bhh, wlin2t, blin2, wfc1t, bfc1):
    full = lambda shape: pl.BlockSpec(shape, lambda: (0,) * len(shape))
    return pl.pallas_call(
        _read_body,
        in_specs=[full((NP, 128)), full((1, NP)), full((NP, 1)),
                  full((128, 128)), full((128, 128)), full((128, 1)),
                  full((128, 1)), full((1, 128)), full((128, 384)),
                  full((128, 384)), full((1, 384)), full((1, 384)),
                  full((128, 128)), full((1, 128)), full((128, 128)),
                  full((1, 128))],
        out_specs=full((NUM_GRAPHS, 128)),
        out_shape=jax.ShapeDtypeStruct((NUM_GRAPHS, 128), jnp.float32),
    )(x, batch_row, batch_col, srct, dstt, asrc, adst, mbias,
      wiht, whht, bih, bhh, wlin2t, blin2, wfc1t, bfc1)


def kernel(x, edge_index, edge_attr, batch,
           w_lin1, b_lin1,
           gate_lin1_w, gate_att_l, gate_att_r, gate_lin2_w, gate_bias,
           gru0_wih, gru0_whh, gru0_bih, gru0_bhh,
           gru1_wih, gru1_whh, gru1_bih, gru1_bhh,
           gru2_wih, gru2_whh, gru2_bih, gru2_bhh,
           gru3_wih, gru3_whh, gru3_bih, gru3_bhh,
           conv1_w, conv1_att_src, conv1_att_dst, conv1_bias,
           conv2_w, conv2_att_src, conv2_att_dst, conv2_bias,
           conv3_w, conv3_att_src, conv3_att_dst, conv3_bias,
           mol_lin_src_w, mol_lin_dst_w, mol_att_src, mol_att_dst, mol_bias,
           molgru_wih, molgru_whh, molgru_bih, molgru_bhh,
           w_lin2, b_lin2, w_fc1, b_fc1):
    f32 = jnp.float32
    x_p = jnp.pad(x, ((0, NP - N_NODES), (0, 0)))
    src_p = jnp.pad(edge_index[0], (0, EP - N_EDGES))
    dst_p = jnp.pad(edge_index[1], (0, EP - N_EDGES))
    ea_p = jnp.pad(edge_attr[:, 0], (0, EP - N_EDGES))

    u1t = gate_lin1_w[:, :128].T
    wcol = gate_lin1_w[:, 128]
    x1, u, y, r = _pre_call(x_p, w_lin1.T, b_lin1[None, :], u1t,
                            gate_lin2_w.T, gate_att_r[:, None])

    e_p = _gate_alpha_call(src_p, dst_p, ea_p, u, r.reshape(NP),
                           wcol, gate_att_l)
    num, den = _scatter_call(False, src_p, dst_p, e_p, e_p, y)

    convs = [(conv1_w, conv1_att_src, conv1_att_dst, conv1_bias),
             (conv2_w, conv2_att_src, conv2_att_dst, conv2_bias),
             (conv3_w, conv3_att_src, conv3_att_dst, conv3_bias)]
    grus = [(gru0_wih, gru0_whh, gru0_bih, gru0_bhh),
            (gru1_wih, gru1_whh, gru1_bih, gru1_bhh),
            (gru2_wih, gru2_whh, gru2_bih, gru2_bhh),
            (gru3_wih, gru3_whh, gru3_bih, gru3_bhh)]

    x_cur = x1
    bias_cur = gate_bias
    for l in range(NUM_LAYERS):
        wih, whh, bih, bhh = grus[l]
        if l < NUM_LAYERS - 1:
            cw, ca_s, ca_d, cb = convs[l]
            x_cur, xt, s, d = _gru_call(
                num, den, bias_cur[None, :], x_cur, wih.T, whh.T,
                bih[None, :], bhh[None, :],
                nxt=(cw.T, ca_s[:, None], ca_d[:, None]))
            num, den = _scatter_call(True, src_p, dst_p,
                                     s.reshape(NP), d.reshape(NP), xt)
            bias_cur = cb
        else:
            (x_cur,) = _gru_call(num, den, bias_cur[None, :], x_cur,
                                 wih.T, whh.T, bih[None, :], bhh[None, :])

    batch_p = jnp.full((NP,), NUM_GRAPHS, jnp.int32).at[:N_NODES].set(batch)
    out = _read_call(
        x_cur, batch_p[None, :], batch_p[:, None],
        mol_lin_src_w.T, mol_lin_dst_w.T, mol_att_src[:, None],
        mol_att_dst[:, None], mol_bias[None, :],
        molgru_wih.T, molgru_whh.T, molgru_bih[None, :], molgru_bhh[None, :],
        w_lin2.T, b_lin2[None, :], w_fc1.T, b_fc1[None, :])
    return out
